# baseline (speedup 1.0000x reference)
import functools

import jax
import jax.numpy as jnp
from jax import lax
from jax.experimental import pallas as pl
from jax.experimental.pallas import tpu as pltpu
from jax.experimental.pallas import tpu_sc as plsc

B, T, D = 1, 2048, 1024
H = 16
HD = D // H
S = 512
K = 8
TB = 256


def _topk_packed(s, n_idx_bits, k, clip, sc_bits):
    sc = jnp.float32(1 << sc_bits)
    q = (jnp.clip(s, -clip, clip) * sc).astype(jnp.int32)
    iota = lax.broadcasted_iota(jnp.int32, s.shape, 0)
    key = (q << n_idx_bits) | iota
    rows = []
    for _ in range(k):
        m2 = jnp.max(key, axis=0, keepdims=True)
        rows.append(m2)
        key = jnp.where(key == m2, jnp.int32(-(2 ** 31)), key)
    packed = jnp.concatenate(rows, axis=0)
    idx = packed & jnp.int32((1 << n_idx_bits) - 1)
    qvals = packed >> n_idx_bits
    return qvals, idx


def _topk_prod(q2, ia, ib, k, sc_bits):
    iota = lax.broadcasted_iota(jnp.int32, q2.shape, 0)
    iota8 = lax.broadcasted_iota(jnp.int32, ia.shape, 0)
    key = (q2 << 6) | iota
    vrows, irows = [], []
    for _ in range(k):
        m2 = jnp.max(key, axis=0, keepdims=True)
        pos = m2 & 63
        i_ = pos >> 3
        j_ = pos & 7
        sel_a = jnp.sum(jnp.where(iota8 == i_, ia, 0), axis=0, keepdims=True)
        sel_b = jnp.sum(jnp.where(iota8 == j_, ib, 0), axis=0, keepdims=True)
        irows.append(sel_a * S + sel_b)
        vrows.append(m2)
        key = jnp.where(key == m2, jnp.int32(-(2 ** 31)), key)
    pv = ((jnp.concatenate(vrows, axis=0) >> 6).astype(jnp.float32)
          * (1.0 / jnp.float32(1 << sc_bits)))
    eidx = jnp.concatenate(irows, axis=0)
    return pv, eidx


def _route_body(xt_ref, win_ref, ka_ref, kb_ref, u_ref, idx_ref, w_ref):
    ht = jnp.dot(win_ref[...], xt_ref[...],
                 preferred_element_type=jnp.float32)
    for hh in range(H):
        hvt = ht[hh * HD:(hh + 1) * HD, :]
        sa = jnp.dot(ka_ref[hh], hvt, preferred_element_type=jnp.float32)
        sb = jnp.dot(kb_ref[hh], hvt, preferred_element_type=jnp.float32)
        qa, ia = _topk_packed(sa, 9, K, 1.9, 21)
        qb, ib = _topk_packed(sb, 9, K, 1.9, 21)
        q2 = jnp.concatenate([qa[i:i + 1, :] + qb for i in range(K)], axis=0)
        pv, eidx = _topk_prod(q2, ia, ib, K, 21)
        m = jnp.max(pv, axis=0, keepdims=True)
        e = jnp.exp(pv - m)
        w = e / jnp.sum(e, axis=0, keepdims=True)
        a = jnp.sum(hvt * u_ref[...], axis=0, keepdims=True)
        act = 1.0 / (1.0 + jnp.exp(-a))
        idx_ref[hh * K:(hh + 1) * K, :] = eidx
        w_ref[hh * K:(hh + 1) * K, :] = w * act


def _route(xt, win, ka, kb, u_col):
    Tn = xt.shape[1]
    grid = (Tn // TB,)
    return pl.pallas_call(
        _route_body,
        grid=grid,
        in_specs=[
            pl.BlockSpec((D, TB), lambda i: (0, i)),
            pl.BlockSpec((D, D), lambda i: (0, 0)),
            pl.BlockSpec((H, S, HD), lambda i: (0, 0, 0)),
            pl.BlockSpec((H, S, HD), lambda i: (0, 0, 0)),
            pl.BlockSpec((HD, 1), lambda i: (0, 0)),
        ],
        out_specs=[
            pl.BlockSpec((H * K, TB), lambda i: (0, i)),
            pl.BlockSpec((H * K, TB), lambda i: (0, i)),
        ],
        out_shape=[
            jax.ShapeDtypeStruct((H * K, Tn), jnp.int32),
            jax.ShapeDtypeStruct((H * K, Tn), jnp.float32),
        ],
    )(xt, win, ka, kb, u_col)


NW = 32
CH = 128
PAIRS = CH // K


SUP = 512
NSTR = SUP // CH


def _make_gather_body(rows_per_w):
    n_sup = rows_per_w // SUP

    def _gather_body(idx_hbm, w_hbm, tab_hbm, out_hbm,
                     idx_all, w_all, rows_v, acc_v, sem0, sem1):
        wid = lax.axis_index("s") * 2 + lax.axis_index("c")
        base = wid * rows_per_w
        sems = (sem0, sem1)
        pltpu.sync_copy(idx_hbm.at[pl.ds(base, rows_per_w)], idx_all)
        pltpu.sync_copy(w_hbm.at[pl.ds(base, rows_per_w)], w_all)

        def fire(su, buf):
            for g in range(NSTR):
                pltpu.async_copy(
                    tab_hbm.at[idx_all.at[pl.ds(su * SUP + g * CH, CH)]],
                    rows_v.at[buf, pl.ds(g * CH, CH)], sems[buf])

        def drain(su, buf):
            for g in range(NSTR):
                pltpu.make_async_copy(
                    tab_hbm.at[idx_all.at[pl.ds(su * SUP + g * CH, CH)]],
                    rows_v.at[buf, pl.ds(g * CH, CH)], sems[buf]).wait()

        def compute(su, buf):
            def pair(p, carry):
                r0 = p * K
                wv = [plsc.load_gather(
                    w_all, [jnp.full((16,), su * SUP + r0 + k, jnp.int32)])
                    for k in range(K)]
                for j in range(HD // 16):
                    acc = None
                    for k in range(K):
                        term = (wv[k]
                                * rows_v[buf, r0 + k, pl.ds(j * 16, 16)])
                        acc = term if acc is None else acc + term
                    acc_v[p, pl.ds(j * 16, 16)] = acc
                return carry

            lax.fori_loop(0, SUP // K, pair, 0, unroll=False)
            pltpu.sync_copy(acc_v, out_hbm.at[pl.ds(wid * (rows_per_w // K)
                                                    + su * (SUP // K),
                                                    SUP // K)])

        fire(0, 0)

        def super2(s2, carry):
            s0 = s2 * 2
            fire(s0 + 1, 1)
            drain(s0, 0)
            compute(s0, 0)

            @pl.when(s0 + 2 < n_sup)
            def _():
                fire(s0 + 2, 0)

            drain(s0 + 1, 1)
            compute(s0 + 1, 1)
            return carry

        lax.fori_loop(0, n_sup // 2, super2, 0, unroll=False)

    return _gather_body


def _sc_gather(flat_idx, flat_w, expert_v):
    n_idx = flat_idx.shape[0]
    rows_per_w = n_idx // NW
    mesh = plsc.VectorSubcoreMesh(core_axis_name="c", subcore_axis_name="s")
    f = pl.kernel(
        _make_gather_body(rows_per_w),
        mesh=mesh,
        out_type=jax.ShapeDtypeStruct((n_idx // K, HD), jnp.float32),
        scratch_types=[
            pltpu.VMEM((rows_per_w,), jnp.int32),
            pltpu.VMEM((rows_per_w,), jnp.float32),
            pltpu.VMEM((2, SUP, HD), jnp.float32),
            pltpu.VMEM((SUP // K, HD), jnp.float32),
            pltpu.SemaphoreType.DMA,
            pltpu.SemaphoreType.DMA,
        ],
        compiler_params=pltpu.CompilerParams(use_tc_tiling_on_sc=False,
                                             needs_layout_passes=False),
    )
    return f(flat_idx, flat_w, expert_v)


def _finish_body(m_ref, wo_ref, g_ref, b_ref, out_ref):
    y = jnp.dot(m_ref[...], wo_ref[...], preferred_element_type=jnp.float32)
    mu = jnp.mean(y, axis=1, keepdims=True)
    yc = y - mu
    var = jnp.mean(yc * yc, axis=1, keepdims=True)
    out_ref[...] = yc * lax.rsqrt(var + 1e-5) * g_ref[...] + b_ref[...]


def _finish(merged, wot, g_row, b_row):
    Tn = merged.shape[0]
    grid = (Tn // TB,)
    return pl.pallas_call(
        _finish_body,
        grid=grid,
        in_specs=[
            pl.BlockSpec((TB, D), lambda i: (i, 0)),
            pl.BlockSpec((D, D), lambda i: (0, 0)),
            pl.BlockSpec((1, D), lambda i: (0, 0)),
            pl.BlockSpec((1, D), lambda i: (0, 0)),
        ],
        out_specs=pl.BlockSpec((TB, D), lambda i: (i, 0)),
        out_shape=jax.ShapeDtypeStruct((Tn, D), jnp.float32),
    )(merged, wot, g_row, b_row)


NSPLIT = 2


def kernel(x, W_in, keys_a, keys_b, u_shared, expert_v, W_out, gamma, beta):
    xt = x.reshape(T, D).T
    u_col = u_shared.reshape(HD, 1)
    wot = W_out.T
    g_row = gamma.reshape(1, D)
    b_row = beta.reshape(1, D)
    T2 = T // NSPLIT
    merged = []
    for s in range(NSPLIT):
        xs = lax.slice_in_dim(xt, s * T2, (s + 1) * T2, axis=1)
        idxt, wt_ = _route(xs, W_in, keys_a, keys_b, u_col)
        merged.append(_sc_gather(idxt.T.reshape(-1), wt_.T.reshape(-1),
                                 expert_v))
    outs = [_finish(m.reshape(T2, D), wot, g_row, b_row) for m in merged]
    return jnp.concatenate(outs, axis=0).reshape(B, T, D)


# speedup vs baseline: 1.0658x; 1.0658x over previous
import functools

import jax
import jax.numpy as jnp
from jax import lax
from jax.experimental import pallas as pl
from jax.experimental.pallas import tpu as pltpu
from jax.experimental.pallas import tpu_sc as plsc

B, T, D = 1, 2048, 1024
H = 16
HD = D // H
S = 512
K = 8
TB = 256


def _topk_packed(s, n_idx_bits, k, clip, sc_bits):
    sc = jnp.float32(1 << sc_bits)
    q = (jnp.clip(s, -clip, clip) * sc).astype(jnp.int32)
    iota = lax.broadcasted_iota(jnp.int32, s.shape, 0)
    key = (q << n_idx_bits) | iota
    half = s.shape[0] // 2
    kt = key[:half, :]
    kb = key[half:, :]
    v1 = jnp.maximum(kt, kb)
    v2 = jnp.minimum(kt, kb)
    rows = []
    for _ in range(k):
        m2 = jnp.max(v1, axis=0, keepdims=True)
        rows.append(m2)
        oh = v1 == m2
        v1 = jnp.where(oh, v2, v1)
        v2 = jnp.where(oh, jnp.int32(-(2 ** 31)), v2)
    packed = jnp.concatenate(rows, axis=0)
    idx = packed & jnp.int32((1 << n_idx_bits) - 1)
    qvals = packed >> n_idx_bits
    return qvals, idx


def _topk_prod(q2, ia, ib, k, sc_bits):
    iota = lax.broadcasted_iota(jnp.int32, q2.shape, 0)
    iota8 = lax.broadcasted_iota(jnp.int32, ia.shape, 0)
    key = (q2 << 6) | iota
    vrows, irows = [], []
    for _ in range(k):
        m2 = jnp.max(key, axis=0, keepdims=True)
        pos = m2 & 63
        i_ = pos >> 3
        j_ = pos & 7
        sel_a = jnp.sum(jnp.where(iota8 == i_, ia, 0), axis=0, keepdims=True)
        sel_b = jnp.sum(jnp.where(iota8 == j_, ib, 0), axis=0, keepdims=True)
        irows.append(sel_a * S + sel_b)
        vrows.append(m2)
        key = jnp.where(key == m2, jnp.int32(-(2 ** 31)), key)
    pv = ((jnp.concatenate(vrows, axis=0) >> 6).astype(jnp.float32)
          * (1.0 / jnp.float32(1 << sc_bits)))
    eidx = jnp.concatenate(irows, axis=0)
    return pv, eidx


def _route_body(xt_ref, win_ref, ka_ref, kb_ref, u_ref, idx_ref, w_ref):
    ht = jnp.dot(win_ref[...], xt_ref[...],
                 preferred_element_type=jnp.float32)
    for hh in range(H):
        hvt = ht[hh * HD:(hh + 1) * HD, :]
        sa = jnp.dot(ka_ref[hh], hvt, preferred_element_type=jnp.float32)
        sb = jnp.dot(kb_ref[hh], hvt, preferred_element_type=jnp.float32)
        qa, ia = _topk_packed(sa, 9, K, 1.9, 21)
        qb, ib = _topk_packed(sb, 9, K, 1.9, 21)
        q2 = jnp.concatenate([qa[i:i + 1, :] + qb for i in range(K)], axis=0)
        pv, eidx = _topk_prod(q2, ia, ib, K, 21)
        m = jnp.max(pv, axis=0, keepdims=True)
        e = jnp.exp(pv - m)
        w = e / jnp.sum(e, axis=0, keepdims=True)
        a = jnp.sum(hvt * u_ref[...], axis=0, keepdims=True)
        act = 1.0 / (1.0 + jnp.exp(-a))
        idx_ref[hh * K:(hh + 1) * K, :] = eidx
        w_ref[hh * K:(hh + 1) * K, :] = w * act


def _route(xt, win, ka, kb, u_col):
    Tn = xt.shape[1]
    grid = (Tn // TB,)
    return pl.pallas_call(
        _route_body,
        grid=grid,
        in_specs=[
            pl.BlockSpec((D, TB), lambda i: (0, i)),
            pl.BlockSpec((D, D), lambda i: (0, 0)),
            pl.BlockSpec((H, S, HD), lambda i: (0, 0, 0)),
            pl.BlockSpec((H, S, HD), lambda i: (0, 0, 0)),
            pl.BlockSpec((HD, 1), lambda i: (0, 0)),
        ],
        out_specs=[
            pl.BlockSpec((H * K, TB), lambda i: (0, i)),
            pl.BlockSpec((H * K, TB), lambda i: (0, i)),
        ],
        out_shape=[
            jax.ShapeDtypeStruct((H * K, Tn), jnp.int32),
            jax.ShapeDtypeStruct((H * K, Tn), jnp.float32),
        ],
    )(xt, win, ka, kb, u_col)


NW = 32
CH = 128
PAIRS = CH // K


SUP = 512
NSTR = SUP // CH


def _make_gather_body(rows_per_w):
    n_sup = rows_per_w // SUP

    def _gather_body(idx_hbm, w_hbm, tab_hbm, out_hbm,
                     idx_all, w_all, rows_v, acc_v, sem0, sem1):
        wid = lax.axis_index("s") * 2 + lax.axis_index("c")
        base = wid * rows_per_w
        sems = (sem0, sem1)
        pltpu.sync_copy(idx_hbm.at[pl.ds(base, rows_per_w)], idx_all)
        pltpu.sync_copy(w_hbm.at[pl.ds(base, rows_per_w)], w_all)

        def fire(su, buf):
            for g in range(NSTR):
                pltpu.async_copy(
                    tab_hbm.at[idx_all.at[pl.ds(su * SUP + g * CH, CH)]],
                    rows_v.at[buf, pl.ds(g * CH, CH)], sems[buf])

        def drain(su, buf):
            for g in range(NSTR):
                pltpu.make_async_copy(
                    tab_hbm.at[idx_all.at[pl.ds(su * SUP + g * CH, CH)]],
                    rows_v.at[buf, pl.ds(g * CH, CH)], sems[buf]).wait()

        def compute(su, buf):
            def pair(p, carry):
                r0 = p * K
                wv = [plsc.load_gather(
                    w_all, [jnp.full((16,), su * SUP + r0 + k, jnp.int32)])
                    for k in range(K)]
                for j in range(HD // 16):
                    acc = None
                    for k in range(K):
                        term = (wv[k]
                                * rows_v[buf, r0 + k, pl.ds(j * 16, 16)])
                        acc = term if acc is None else acc + term
                    acc_v[p, pl.ds(j * 16, 16)] = acc
                return carry

            lax.fori_loop(0, SUP // K, pair, 0, unroll=False)
            pltpu.sync_copy(acc_v, out_hbm.at[pl.ds(wid * (rows_per_w // K)
                                                    + su * (SUP // K),
                                                    SUP // K)])

        fire(0, 0)

        def super2(s2, carry):
            s0 = s2 * 2
            fire(s0 + 1, 1)
            drain(s0, 0)
            compute(s0, 0)

            @pl.when(s0 + 2 < n_sup)
            def _():
                fire(s0 + 2, 0)

            drain(s0 + 1, 1)
            compute(s0 + 1, 1)
            return carry

        lax.fori_loop(0, n_sup // 2, super2, 0, unroll=False)

    return _gather_body


def _sc_gather(flat_idx, flat_w, expert_v):
    n_idx = flat_idx.shape[0]
    rows_per_w = n_idx // NW
    mesh = plsc.VectorSubcoreMesh(core_axis_name="c", subcore_axis_name="s")
    f = pl.kernel(
        _make_gather_body(rows_per_w),
        mesh=mesh,
        out_type=jax.ShapeDtypeStruct((n_idx // K, HD), jnp.float32),
        scratch_types=[
            pltpu.VMEM((rows_per_w,), jnp.int32),
            pltpu.VMEM((rows_per_w,), jnp.float32),
            pltpu.VMEM((2, SUP, HD), jnp.float32),
            pltpu.VMEM((SUP // K, HD), jnp.float32),
            pltpu.SemaphoreType.DMA,
            pltpu.SemaphoreType.DMA,
        ],
        compiler_params=pltpu.CompilerParams(use_tc_tiling_on_sc=False,
                                             needs_layout_passes=False),
    )
    return f(flat_idx, flat_w, expert_v)


def _finish_body(m_ref, wo_ref, g_ref, b_ref, out_ref):
    y = jnp.dot(m_ref[...], wo_ref[...], preferred_element_type=jnp.float32)
    mu = jnp.mean(y, axis=1, keepdims=True)
    yc = y - mu
    var = jnp.mean(yc * yc, axis=1, keepdims=True)
    out_ref[...] = yc * lax.rsqrt(var + 1e-5) * g_ref[...] + b_ref[...]


def _finish(merged, wot, g_row, b_row):
    Tn = merged.shape[0]
    grid = (Tn // TB,)
    return pl.pallas_call(
        _finish_body,
        grid=grid,
        in_specs=[
            pl.BlockSpec((TB, D), lambda i: (i, 0)),
            pl.BlockSpec((D, D), lambda i: (0, 0)),
            pl.BlockSpec((1, D), lambda i: (0, 0)),
            pl.BlockSpec((1, D), lambda i: (0, 0)),
        ],
        out_specs=pl.BlockSpec((TB, D), lambda i: (i, 0)),
        out_shape=jax.ShapeDtypeStruct((Tn, D), jnp.float32),
    )(merged, wot, g_row, b_row)


NSPLIT = 2


def kernel(x, W_in, keys_a, keys_b, u_shared, expert_v, W_out, gamma, beta):
    xt = x.reshape(T, D).T
    u_col = u_shared.reshape(HD, 1)
    wot = W_out.T
    g_row = gamma.reshape(1, D)
    b_row = beta.reshape(1, D)
    T2 = T // NSPLIT
    merged = []
    for s in range(NSPLIT):
        xs = lax.slice_in_dim(xt, s * T2, (s + 1) * T2, axis=1)
        idxt, wt_ = _route(xs, W_in, keys_a, keys_b, u_col)
        merged.append(_sc_gather(idxt.T.reshape(-1), wt_.T.reshape(-1),
                                 expert_v))
    outs = [_finish(m.reshape(T2, D), wot, g_row, b_row) for m in merged]
    return jnp.concatenate(outs, axis=0).reshape(B, T, D)


# speedup vs baseline: 1.1423x; 1.0718x over previous
import functools

import jax
import jax.numpy as jnp
from jax import lax
from jax.experimental import pallas as pl
from jax.experimental.pallas import tpu as pltpu
from jax.experimental.pallas import tpu_sc as plsc

B, T, D = 1, 2048, 1024
H = 16
HD = D // H
S = 512
K = 8
TB = 256


def _topk_packed(s, n_idx_bits, k, clip, sc_bits):
    sc = jnp.float32(1 << sc_bits)
    q = (jnp.clip(s, -clip, clip) * sc).astype(jnp.int32)
    iota = lax.broadcasted_iota(jnp.int32, s.shape, 0)
    key = (q << n_idx_bits) | iota
    half = s.shape[0] // 2
    kt = key[:half, :]
    kb = key[half:, :]
    v1 = jnp.maximum(kt, kb)
    v2 = jnp.minimum(kt, kb)
    rows = []
    for _ in range(k):
        m2 = jnp.max(v1, axis=0, keepdims=True)
        rows.append(m2)
        oh = v1 == m2
        v1 = jnp.where(oh, v2, v1)
        v2 = jnp.where(oh, jnp.int32(-(2 ** 31)), v2)
    packed = jnp.concatenate(rows, axis=0)
    idx = packed & jnp.int32((1 << n_idx_bits) - 1)
    qvals = packed >> n_idx_bits
    return qvals, idx


def _topk_prod(q2, ia, ib, k, sc_bits):
    iota = lax.broadcasted_iota(jnp.int32, q2.shape, 0)
    iota8 = lax.broadcasted_iota(jnp.int32, ia.shape, 0)
    key = (q2 << 6) | iota
    vrows, irows = [], []
    for _ in range(k):
        m2 = jnp.max(key, axis=0, keepdims=True)
        pos = m2 & 63
        i_ = pos >> 3
        j_ = pos & 7
        sel_a = jnp.sum(jnp.where(iota8 == i_, ia, 0), axis=0, keepdims=True)
        sel_b = jnp.sum(jnp.where(iota8 == j_, ib, 0), axis=0, keepdims=True)
        irows.append(sel_a * S + sel_b)
        vrows.append(m2)
        key = jnp.where(key == m2, jnp.int32(-(2 ** 31)), key)
    pv = ((jnp.concatenate(vrows, axis=0) >> 6).astype(jnp.float32)
          * (1.0 / jnp.float32(1 << sc_bits)))
    eidx = jnp.concatenate(irows, axis=0)
    return pv, eidx


def _route_body(xt_ref, win_ref, ka_ref, kb_ref, u_ref, idx_ref, w_ref):
    ht = jnp.dot(win_ref[...], xt_ref[...],
                 preferred_element_type=jnp.float32)
    for hh in range(H):
        hvt = ht[hh * HD:(hh + 1) * HD, :]
        sa = jnp.dot(ka_ref[hh], hvt, preferred_element_type=jnp.float32)
        sb = jnp.dot(kb_ref[hh], hvt, preferred_element_type=jnp.float32)
        qa, ia = _topk_packed(sa, 9, K, 1.9, 21)
        qb, ib = _topk_packed(sb, 9, K, 1.9, 21)
        q2 = jnp.concatenate([qa[i:i + 1, :] + qb for i in range(K)], axis=0)
        pv, eidx = _topk_prod(q2, ia, ib, K, 21)
        m = jnp.max(pv, axis=0, keepdims=True)
        e = jnp.exp(pv - m)
        w = e / jnp.sum(e, axis=0, keepdims=True)
        a = jnp.sum(hvt * u_ref[...], axis=0, keepdims=True)
        act = 1.0 / (1.0 + jnp.exp(-a))
        idx_ref[hh * K:(hh + 1) * K, :] = eidx
        w_ref[hh * K:(hh + 1) * K, :] = w * act


def _route(xt, win, ka, kb, u_col):
    Tn = xt.shape[1]
    grid = (Tn // TB,)
    return pl.pallas_call(
        _route_body,
        grid=grid,
        in_specs=[
            pl.BlockSpec((D, TB), lambda i: (0, i)),
            pl.BlockSpec((D, D), lambda i: (0, 0)),
            pl.BlockSpec((H, S, HD), lambda i: (0, 0, 0)),
            pl.BlockSpec((H, S, HD), lambda i: (0, 0, 0)),
            pl.BlockSpec((HD, 1), lambda i: (0, 0)),
        ],
        out_specs=[
            pl.BlockSpec((H * K, TB), lambda i: (0, i)),
            pl.BlockSpec((H * K, TB), lambda i: (0, i)),
        ],
        out_shape=[
            jax.ShapeDtypeStruct((H * K, Tn), jnp.int32),
            jax.ShapeDtypeStruct((H * K, Tn), jnp.float32),
        ],
    )(xt, win, ka, kb, u_col)


NW = 32
CH = 128
PAIRS = CH // K


SUP = 512
NSTR = SUP // CH


def _make_gather_body(rows_per_w):
    n_sup = rows_per_w // SUP

    def _gather_body(idx_hbm, w_hbm, tab_hbm, out_hbm,
                     idx_all, w_all, rows_v, acc_v, sem0, sem1):
        wid = lax.axis_index("s") * 2 + lax.axis_index("c")
        base = wid * rows_per_w
        sems = (sem0, sem1)
        pltpu.sync_copy(idx_hbm.at[pl.ds(base, rows_per_w)], idx_all)
        pltpu.sync_copy(w_hbm.at[pl.ds(base, rows_per_w)], w_all)

        def fire(su, buf):
            for g in range(NSTR):
                pltpu.async_copy(
                    tab_hbm.at[idx_all.at[pl.ds(su * SUP + g * CH, CH)]],
                    rows_v.at[buf, pl.ds(g * CH, CH)], sems[buf])

        def drain(su, buf):
            for g in range(NSTR):
                pltpu.make_async_copy(
                    tab_hbm.at[idx_all.at[pl.ds(su * SUP + g * CH, CH)]],
                    rows_v.at[buf, pl.ds(g * CH, CH)], sems[buf]).wait()

        def compute(su, buf):
            def pair(p, carry):
                r0 = p * K
                wv = [plsc.load_gather(
                    w_all, [jnp.full((16,), su * SUP + r0 + k, jnp.int32)])
                    for k in range(K)]
                for j in range(HD // 16):
                    acc = None
                    for k in range(K):
                        term = (wv[k]
                                * rows_v[buf, r0 + k, pl.ds(j * 16, 16)])
                        acc = term if acc is None else acc + term
                    acc_v[p, pl.ds(j * 16, 16)] = acc
                return carry

            lax.fori_loop(0, SUP // K, pair, 0, unroll=False)
            pltpu.sync_copy(acc_v, out_hbm.at[pl.ds(wid * (rows_per_w // K)
                                                    + su * (SUP // K),
                                                    SUP // K)])

        fire(0, 0)

        def super2(s2, carry):
            s0 = s2 * 2
            fire(s0 + 1, 1)
            drain(s0, 0)
            compute(s0, 0)

            @pl.when(s0 + 2 < n_sup)
            def _():
                fire(s0 + 2, 0)

            drain(s0 + 1, 1)
            compute(s0 + 1, 1)
            return carry

        lax.fori_loop(0, n_sup // 2, super2, 0, unroll=False)

    return _gather_body


def _sc_gather(flat_idx, flat_w, expert_v):
    n_idx = flat_idx.shape[0]
    rows_per_w = n_idx // NW
    mesh = plsc.VectorSubcoreMesh(core_axis_name="c", subcore_axis_name="s")
    f = pl.kernel(
        _make_gather_body(rows_per_w),
        mesh=mesh,
        out_type=jax.ShapeDtypeStruct((n_idx // K, HD), jnp.float32),
        scratch_types=[
            pltpu.VMEM((rows_per_w,), jnp.int32),
            pltpu.VMEM((rows_per_w,), jnp.float32),
            pltpu.VMEM((2, SUP, HD), jnp.float32),
            pltpu.VMEM((SUP // K, HD), jnp.float32),
            pltpu.SemaphoreType.DMA,
            pltpu.SemaphoreType.DMA,
        ],
        compiler_params=pltpu.CompilerParams(use_tc_tiling_on_sc=False,
                                             needs_layout_passes=False),
    )
    return f(flat_idx, flat_w, expert_v)


def _finish_body(m_ref, wo_ref, g_ref, b_ref, out_ref):
    y = jnp.dot(m_ref[...], wo_ref[...], preferred_element_type=jnp.float32)
    mu = jnp.mean(y, axis=1, keepdims=True)
    yc = y - mu
    var = jnp.mean(yc * yc, axis=1, keepdims=True)
    out_ref[...] = yc * lax.rsqrt(var + 1e-5) * g_ref[...] + b_ref[...]


def _finish(merged, wot, g_row, b_row):
    Tn = merged.shape[0]
    grid = (Tn // TB,)
    return pl.pallas_call(
        _finish_body,
        grid=grid,
        in_specs=[
            pl.BlockSpec((TB, D), lambda i: (i, 0)),
            pl.BlockSpec((D, D), lambda i: (0, 0)),
            pl.BlockSpec((1, D), lambda i: (0, 0)),
            pl.BlockSpec((1, D), lambda i: (0, 0)),
        ],
        out_specs=pl.BlockSpec((TB, D), lambda i: (i, 0)),
        out_shape=jax.ShapeDtypeStruct((Tn, D), jnp.float32),
    )(merged, wot, g_row, b_row)


NSPLIT = 1


def kernel(x, W_in, keys_a, keys_b, u_shared, expert_v, W_out, gamma, beta):
    xt = x.reshape(T, D).T
    u_col = u_shared.reshape(HD, 1)
    wot = W_out.T
    g_row = gamma.reshape(1, D)
    b_row = beta.reshape(1, D)
    T2 = T // NSPLIT
    merged = []
    for s in range(NSPLIT):
        xs = lax.slice_in_dim(xt, s * T2, (s + 1) * T2, axis=1)
        idxt, wt_ = _route(xs, W_in, keys_a, keys_b, u_col)
        merged.append(_sc_gather(idxt.T.reshape(-1), wt_.T.reshape(-1),
                                 expert_v))
    outs = [_finish(m.reshape(T2, D), wot, g_row, b_row) for m in merged]
    return jnp.concatenate(outs, axis=0).reshape(B, T, D)
